# Initial kernel scaffold; baseline (speedup 1.0000x reference)
import jax
import jax.numpy as jnp
from jax.experimental import pallas as pl


def kernel(hidden_states, cos, sin, g1, g2, Wq, bq, Wk, bk, Wv, bv, Wo, Wr, Wg, Wu, Wd):
    raise NotImplementedError("write your pallas kernel here")


# speedup vs baseline: 1.1295x; 1.1295x over previous
import jax
import jax.numpy as jnp
from jax.experimental import pallas as pl
from jax.experimental.pallas import tpu as pltpu

_B, _S, _D = 1, 2048, 1024
_H, _Dh = 16, 64
_E, _K, _F = 8, 2, 1408
_EPS = 1e-06
_JITTER = 0.01
_TB = 256
_NTB = _S // _TB


def _rms(x, g):
    v = jnp.mean(x * x, axis=-1, keepdims=True)
    return x * jax.lax.rsqrt(v + _EPS) * g


def _mm(a, b_t, out_dtype=jnp.float32):
    return jax.lax.dot_general(a, b_t, (((1,), (1,)), ((), ())),
                               preferred_element_type=out_dtype)


def _qkv_kernel(hs_ref, g1_ref, wq_ref, wk_ref, wv_ref, bq_ref, bk_ref,
                bv_ref, cos_ref, sin_ref, q_ref, k_ref, v_ref):
    x = hs_ref[...]
    h = _rms(x, g1_ref[...])
    c = cos_ref[...][:, None, :]
    s = sin_ref[...][:, None, :]
    sgn = jnp.where(
        jax.lax.broadcasted_iota(jnp.int32, (1, 1, _Dh), 2) < (_Dh // 2),
        -1.0, 1.0).astype(jnp.float32)

    def rope(y):
        y3 = y.reshape(_TB, _H, _Dh)
        rot = jnp.roll(y3, _Dh // 2, axis=-1) * sgn
        return (y3 * c + rot * s).transpose(1, 0, 2)

    q = _mm(h, wq_ref[...]) + bq_ref[...]
    k = _mm(h, wk_ref[...]) + bk_ref[...]
    v = _mm(h, wv_ref[...]) + bv_ref[...]
    q_ref[...] = rope(q)
    k_ref[...] = rope(k)
    v_ref[...] = v.reshape(_TB, _H, _Dh).transpose(1, 0, 2)


def _attn_kernel(q_ref, k_ref, v_ref, o_ref):
    i = pl.program_id(1)
    q = q_ref[0]
    k = k_ref[0]
    v = v_ref[0]
    s = _mm(q, k) * (1.0 / (_Dh ** 0.5))
    row = i * _TB + jax.lax.broadcasted_iota(jnp.int32, (_TB, _S), 0)
    col = jax.lax.broadcasted_iota(jnp.int32, (_TB, _S), 1)
    s = jnp.where(col <= row, s, jnp.float32(-1e9))
    m = jnp.max(s, axis=-1, keepdims=True)
    p = jnp.exp(s - m)
    l = jnp.sum(p, axis=-1, keepdims=True)
    a = p / l
    o_ref[0] = jax.lax.dot_general(a, v, (((1,), (0,)), ((), ())),
                                   preferred_element_type=jnp.float32)


def _post_kernel(ctx_ref, res_ref, wo_ref, g2_ref, wr_ref, noise_ref,
                 h_ref, x_ref, disp_ref, aux_ref):
    i = pl.program_id(0)
    attn_out = _mm(ctx_ref[...], wo_ref[...])
    h = attn_out + res_ref[...]
    h_ref[...] = h
    x = _rms(h, g2_ref[...])
    x_ref[...] = x.astype(jnp.bfloat16)
    logits = _mm(x, wr_ref[...]) + noise_ref[...]
    m = jnp.max(logits, axis=-1, keepdims=True)
    p = jnp.exp(logits - m)
    probs = p / jnp.sum(p, axis=-1, keepdims=True)
    lane = jax.lax.broadcasted_iota(jnp.int32, (_TB, _E), 1)
    i1 = jnp.argmax(probs, axis=-1)[:, None]
    oh1 = (lane == i1)
    v1 = jnp.max(probs, axis=-1, keepdims=True)
    probs2 = jnp.where(oh1, -1.0, probs)
    i2 = jnp.argmax(probs2, axis=-1)[:, None]
    oh2 = (lane == i2)
    v2 = jnp.max(probs2, axis=-1, keepdims=True)
    wsum = v1 + v2
    disp_ref[...] = (jnp.where(oh1, v1, 0.0) + jnp.where(oh2, v2, 0.0)) / wsum

    part = jnp.sum(probs, axis=0, keepdims=True)

    @pl.when(i == 0)
    def _():
        aux_ref[...] = part

    @pl.when(i > 0)
    def _():
        aux_ref[...] += part


def _moe_kernel(x_ref, wg_ref, wu_ref, wd_ref, disp_ref, h_ref,
                out_ref, acc_ref):
    e = pl.program_id(0)
    i = pl.program_id(1)
    x = x_ref[...]
    g = _mm(x, wg_ref[0])
    u = _mm(x, wu_ref[0])
    a = (g * jax.lax.logistic(g) * u).astype(jnp.bfloat16)
    pe = _mm(a, wd_ref[0])
    lane = jax.lax.broadcasted_iota(jnp.int32, (_TB, _E), 1)
    w = jnp.sum(jnp.where(lane == e, disp_ref[...], 0.0), axis=-1,
                keepdims=True)
    contrib = pe * w
    rows = pl.ds(i * _TB, _TB)

    @pl.when(e == 0)
    def _():
        acc_ref[rows, :] = h_ref[...] + contrib

    @pl.when(e > 0)
    def _():
        acc_ref[rows, :] += contrib

    @pl.when(e == _E - 1)
    def _():
        out_ref[...] = acc_ref[rows, :]


def kernel(hidden_states, cos, sin, g1, g2, Wq, bq, Wk, bk, Wv, bv, Wo,
           Wr, Wg, Wu, Wd):
    hs = hidden_states.reshape(_S, _D)
    cos2 = cos.reshape(_S, _Dh)
    sin2 = sin.reshape(_S, _Dh)
    g1r = g1.reshape(1, _D)
    g2r = g2.reshape(1, _D)
    bqr = bq.reshape(1, _D)
    bkr = bk.reshape(1, _D)
    bvr = bv.reshape(1, _D)
    noise = (jax.random.normal(jax.random.key(42), (_S, _E), jnp.float32)
             * _JITTER)

    f32 = jnp.float32
    qkv_shapes = [jax.ShapeDtypeStruct((_H, _S, _Dh), f32)] * 3
    q, k, v = pl.pallas_call(
        _qkv_kernel,
        grid=(_NTB,),
        in_specs=[
            pl.BlockSpec((_TB, _D), lambda i: (i, 0)),
            pl.BlockSpec((1, _D), lambda i: (0, 0)),
            pl.BlockSpec((_D, _D), lambda i: (0, 0)),
            pl.BlockSpec((_D, _D), lambda i: (0, 0)),
            pl.BlockSpec((_D, _D), lambda i: (0, 0)),
            pl.BlockSpec((1, _D), lambda i: (0, 0)),
            pl.BlockSpec((1, _D), lambda i: (0, 0)),
            pl.BlockSpec((1, _D), lambda i: (0, 0)),
            pl.BlockSpec((_TB, _Dh), lambda i: (i, 0)),
            pl.BlockSpec((_TB, _Dh), lambda i: (i, 0)),
        ],
        out_specs=[pl.BlockSpec((_H, _TB, _Dh), lambda i: (0, i, 0))] * 3,
        out_shape=qkv_shapes,
    )(hs, g1r, Wq, Wk, Wv, bqr, bkr, bvr, cos2, sin2)

    ctx = pl.pallas_call(
        _attn_kernel,
        grid=(_H, _NTB),
        in_specs=[
            pl.BlockSpec((1, _TB, _Dh), lambda h, i: (h, i, 0)),
            pl.BlockSpec((1, _S, _Dh), lambda h, i: (h, 0, 0)),
            pl.BlockSpec((1, _S, _Dh), lambda h, i: (h, 0, 0)),
        ],
        out_specs=pl.BlockSpec((1, _TB, _Dh), lambda h, i: (h, i, 0)),
        out_shape=jax.ShapeDtypeStruct((_H, _S, _Dh), f32),
    )(q, k, v)

    ctx2d = ctx.transpose(1, 0, 2).reshape(_S, _D)

    h_res, x_bf, disp, aux_part = pl.pallas_call(
        _post_kernel,
        grid=(_NTB,),
        in_specs=[
            pl.BlockSpec((_TB, _D), lambda i: (i, 0)),
            pl.BlockSpec((_TB, _D), lambda i: (i, 0)),
            pl.BlockSpec((_D, _D), lambda i: (0, 0)),
            pl.BlockSpec((1, _D), lambda i: (0, 0)),
            pl.BlockSpec((_E, _D), lambda i: (0, 0)),
            pl.BlockSpec((_TB, _E), lambda i: (i, 0)),
        ],
        out_specs=[
            pl.BlockSpec((_TB, _D), lambda i: (i, 0)),
            pl.BlockSpec((_TB, _D), lambda i: (i, 0)),
            pl.BlockSpec((_TB, _E), lambda i: (i, 0)),
            pl.BlockSpec((1, _E), lambda i: (0, 0)),
        ],
        out_shape=[
            jax.ShapeDtypeStruct((_S, _D), f32),
            jax.ShapeDtypeStruct((_S, _D), jnp.bfloat16),
            jax.ShapeDtypeStruct((_S, _E), f32),
            jax.ShapeDtypeStruct((1, _E), f32),
        ],
    )(ctx2d, hs, Wo, g2r, Wr, noise)

    wg_b = Wg.astype(jnp.bfloat16)
    wu_b = Wu.astype(jnp.bfloat16)
    wd_b = Wd.astype(jnp.bfloat16)

    out2d = pl.pallas_call(
        _moe_kernel,
        grid=(_E, _NTB),
        in_specs=[
            pl.BlockSpec((_TB, _D), lambda e, i: (i, 0)),
            pl.BlockSpec((1, _F, _D), lambda e, i: (e, 0, 0)),
            pl.BlockSpec((1, _F, _D), lambda e, i: (e, 0, 0)),
            pl.BlockSpec((1, _D, _F), lambda e, i: (e, 0, 0)),
            pl.BlockSpec((_TB, _E), lambda e, i: (i, 0)),
            pl.BlockSpec((_TB, _D), lambda e, i: (i, 0)),
        ],
        out_specs=pl.BlockSpec((_TB, _D), lambda e, i: (i, 0)),
        out_shape=jax.ShapeDtypeStruct((_S, _D), f32),
        scratch_shapes=[pltpu.VMEM((_S, _D), f32)],
    )(x_bf, wg_b, wu_b, wd_b, disp, h_res)

    aux_loss = jnp.mean(_E * (aux_part[0] / _S) ** 2)
    return out2d.reshape(_B, _S, _D), aux_loss
